# Initial kernel scaffold; baseline (speedup 1.0000x reference)
import jax
import jax.numpy as jnp
from jax.experimental import pallas as pl


def kernel(x_s, x_t, edge_attr_s, edge_attr_t, edge_index_s, edge_index_t, x_s_batch, x_t_batch, y, node_emb, edge_emb, W_in_0, W_in_1, W_out_0, W_out_1, gin_W, gin_b, gout_W, gout_b, cls_W1, cls_b1, cls_W2, cls_b2):
    raise NotImplementedError("write your pallas kernel here")


# speedup vs baseline: 1.0686x; 1.0686x over previous
import functools

import jax
import jax.numpy as jnp
from jax.experimental import pallas as pl
from jax.experimental.pallas import tpu as pltpu

_L = 2


def _matmul_body(x_ref, w_ref, o_ref):
    o_ref[...] = jnp.dot(x_ref[...], w_ref[...],
                         preferred_element_type=jnp.float32)


def _matmul(x, w, bn):
    n, k = x.shape
    m = w.shape[1]
    return pl.pallas_call(
        _matmul_body,
        grid=(n // bn,),
        in_specs=[pl.BlockSpec((bn, k), lambda i: (i, 0)),
                  pl.BlockSpec((k, m), lambda i: (0, 0))],
        out_specs=pl.BlockSpec((bn, m), lambda i: (i, 0)),
        out_shape=jax.ShapeDtypeStruct((n, m), jnp.float32),
    )(x, w)


def kernel(x_s, x_t, edge_attr_s, edge_attr_t, edge_index_s, edge_index_t,
           x_s_batch, x_t_batch, y, node_emb, edge_emb, W_in_0, W_in_1,
           W_out_0, W_out_1, gin_W, gin_b, gout_W, gout_b, cls_W1, cls_b1,
           cls_W2, cls_b2):
    def init(x, table):
        return table[jnp.argmax(x, axis=1)]

    def layer(x, ea, row, col, Wi0, Wi1, Wo0, Wo1, giW, gib, goW, gob):
        N = x.shape[0]
        E = row.shape[0]
        ea_in, ea_out = ea[:E], ea[E:]
        Wr = jnp.concatenate([giW, Wi0[:128], Wo0[144:]], axis=1)
        Wc = jnp.concatenate([goW, Wi0[144:], Wo0[:128]], axis=1)
        Rn = _matmul(x, Wr, 1000)
        Cn = _matmul(x, Wc, 1000)
        Rg = Rn[row]
        Cg = Cn[col]
        ea_in2 = Rg[:, 128:144] + ea_in @ Wi0[128:144] + Cg[:, 128:144]
        ea_out2 = Cg[:, 144:160] + ea_out @ Wo0[128:144] + Rg[:, 144:160]
        w_in = jax.nn.sigmoid(ea_in2 @ Wi1).reshape(-1)
        w_out = jax.nn.sigmoid(ea_out2 @ Wo1).reshape(-1)
        x_in = jax.ops.segment_sum(w_in[:, None] * Rg[:, :128], col,
                                   num_segments=N) + gib
        x_out = jax.ops.segment_sum(w_out[:, None] * Cg[:, :128], row,
                                    num_segments=N) + gob
        return x + x_in + x_out, jnp.concatenate([ea_in2, ea_out2], axis=0)

    def dag(x, ea, ei):
        row, col = ei[0], ei[1]
        for i in range(_L):
            x, ea = layer(x, ea, row, col, W_in_0[i], W_in_1[i], W_out_0[i],
                          W_out_1[i], gin_W[i], gin_b[i], gout_W[i],
                          gout_b[i])
        return x

    h_s = dag(init(x_s, node_emb), init(edge_attr_s, edge_emb), edge_index_s)
    h_t = dag(init(x_t, node_emb), init(edge_attr_t, edge_emb), edge_index_t)

    B = y.shape[0]

    def pool(x, batch_ids):
        sums = jax.ops.segment_sum(x, batch_ids, num_segments=B)
        cnt = jax.ops.segment_sum(jnp.ones((x.shape[0],), x.dtype), batch_ids,
                                  num_segments=B)
        return sums / jnp.maximum(cnt, 1.0)[:, None]

    g_s = pool(h_s, x_s_batch)
    g_t = pool(h_t, x_t_batch)
    z = jnp.concatenate([g_s, g_t], axis=1)
    h = jax.nn.relu(z @ cls_W1 + cls_b1)
    pred = h @ cls_W2 + cls_b2
    logp = jax.nn.log_softmax(pred, axis=1)
    loss = -jnp.mean(logp[jnp.arange(B), y])
    return loss


# speedup vs baseline: 1.6130x; 1.5095x over previous
import functools

import jax
import jax.numpy as jnp
from jax import lax
from jax.experimental import pallas as pl
from jax.experimental.pallas import tpu as pltpu
from jax.experimental.pallas import tpu_sc as plsc

_L = 2
_N = 10000
_E = 160000
_NPAD = 10112
_NT = 32
_EPT = 5120
_EPAD = _NT * _EPT
_K = 64
_NCH = _EPT // _K
_RPS = _NPAD // 16


def _matmul_body(x_ref, w_ref, o_ref):
    o_ref[...] = jnp.dot(x_ref[...], w_ref[...],
                         preferred_element_type=jnp.float32)


def _matmul(x, w, bn):
    n, k = x.shape
    m = w.shape[1]
    return pl.pallas_call(
        _matmul_body,
        grid=(n // bn,),
        in_specs=[pl.BlockSpec((bn, k), lambda i: (i, 0)),
                  pl.BlockSpec((k, m), lambda i: (0, 0))],
        out_specs=pl.BlockSpec((bn, m), lambda i: (i, 0)),
        out_shape=jax.ShapeDtypeStruct((n, m), jnp.float32),
    )(x, w)


def _proj_body(x_ref, acc_ref, b_ref, w_ref, xn_ref, r_ref, c_ref):
    xn = (x_ref[...] + acc_ref[0] + acc_ref[1] + b_ref[...])
    xn_ref[...] = xn
    rc = jnp.dot(xn, w_ref[...], preferred_element_type=jnp.float32)
    r_ref[...] = rc[:, :160]
    c_ref[...] = rc[:, 160:]


def _proj(x, acc2, bias, w):
    bn = 632
    grid = (_NPAD // bn,)
    return pl.pallas_call(
        _proj_body,
        grid=grid,
        in_specs=[pl.BlockSpec((bn, 128), lambda i: (i, 0)),
                  pl.BlockSpec((2, bn, 128), lambda i: (0, i, 0)),
                  pl.BlockSpec((1, 128), lambda i: (0, 0)),
                  pl.BlockSpec((128, 320), lambda i: (0, 0))],
        out_specs=[pl.BlockSpec((bn, 128), lambda i: (i, 0)),
                   pl.BlockSpec((bn, 160), lambda i: (i, 0)),
                   pl.BlockSpec((bn, 160), lambda i: (i, 0))],
        out_shape=[jax.ShapeDtypeStruct((_NPAD, 128), jnp.float32),
                   jax.ShapeDtypeStruct((_NPAD, 160), jnp.float32),
                   jax.ShapeDtypeStruct((_NPAD, 160), jnp.float32)],
    )(x, acc2.reshape(2, _NPAD, 128), bias.reshape(1, 128), w)


def _sc_body(Rh, Ch, rih, cih, ein_h, eout_h, wih, woh, zh,
             acc_h, a2_h, b2_h,
             acc_s, ridx_v, cidx_v, Rg, Cg, ein_v, eout_v, min_v, mout_v,
             av, bv, wiv, wov, sem1, sem2):
    c = lax.axis_index("c")
    s = lax.axis_index("s")
    w = c * 16 + s
    pltpu.sync_copy(zh.at[pl.ds(s * _RPS, _RPS)],
                    acc_s.at[pl.ds(s * _RPS, _RPS)])
    pltpu.sync_copy(wih, wiv)
    pltpu.sync_copy(woh, wov)
    plsc.subcore_barrier()
    base = w * _EPT
    wir = wiv[...]
    wor = wov[...]

    def chunk(i, carry):
        off = base + i * _K
        pltpu.sync_copy(rih.at[pl.ds(off, _K)], ridx_v)
        pltpu.sync_copy(cih.at[pl.ds(off, _K)], cidx_v)
        cp1 = pltpu.async_copy(Rh.at[ridx_v], Rg, sem1)
        cp2 = pltpu.async_copy(Ch.at[cidx_v], Cg, sem2)
        pltpu.sync_copy(ein_h.at[pl.ds(off, _K)], ein_v)
        pltpu.sync_copy(eout_h.at[pl.ds(off, _K)], eout_v)
        cp1.wait()
        cp2.wait()

        def edge(e, carry2):
            a = Rg[e, pl.ds(128, 16)] + ein_v[e, :] + Cg[e, pl.ds(128, 16)]
            b = Cg[e, pl.ds(144, 16)] + eout_v[e, :] + Rg[e, pl.ds(144, 16)]
            av[e, :] = a
            bv[e, :] = b
            zi = jnp.broadcast_to(jnp.sum(a * wir, axis=0), (16,))
            zo = jnp.broadcast_to(jnp.sum(b * wor, axis=0), (16,))
            gi = 1.0 / (1.0 + jnp.exp(-zi))
            go = 1.0 / (1.0 + jnp.exp(-zo))
            for j in range(8):
                min_v[e, pl.ds(j * 16, 16)] = gi * Rg[e, pl.ds(j * 16, 16)]
                mout_v[e, pl.ds(j * 16, 16)] = go * Cg[e, pl.ds(j * 16, 16)]
            return carry2

        lax.fori_loop(0, _K, edge, 0)
        pltpu.sync_copy(min_v, acc_s.at[cidx_v], add=True)
        pltpu.sync_copy(mout_v, acc_s.at[ridx_v], add=True)
        pltpu.sync_copy(av, a2_h.at[pl.ds(off, _K)])
        pltpu.sync_copy(bv, b2_h.at[pl.ds(off, _K)])
        return carry

    lax.fori_loop(0, _NCH, chunk, 0)
    plsc.subcore_barrier()
    pltpu.sync_copy(acc_s.at[pl.ds(s * _RPS, _RPS)],
                    acc_h.at[pl.ds(c * _NPAD + s * _RPS, _RPS)])


def _sc_sparse(Rn, Cn, ridx, cidx, eawin, eawout, wi1, wo1, zeros_nd):
    mesh = plsc.VectorSubcoreMesh(core_axis_name="c", subcore_axis_name="s")
    f = pl.kernel(
        _sc_body,
        out_type=(jax.ShapeDtypeStruct((2 * _NPAD, 128), jnp.float32),
                  jax.ShapeDtypeStruct((_EPAD, 16), jnp.float32),
                  jax.ShapeDtypeStruct((_EPAD, 16), jnp.float32)),
        mesh=mesh,
        compiler_params=pltpu.CompilerParams(needs_layout_passes=False,
                                             use_tc_tiling_on_sc=False),
        scratch_types=[
            pltpu.VMEM_SHARED((_NPAD, 128), jnp.float32),
            pltpu.VMEM((_K,), jnp.int32),
            pltpu.VMEM((_K,), jnp.int32),
            pltpu.VMEM((_K, 160), jnp.float32),
            pltpu.VMEM((_K, 160), jnp.float32),
            pltpu.VMEM((_K, 16), jnp.float32),
            pltpu.VMEM((_K, 16), jnp.float32),
            pltpu.VMEM((_K, 128), jnp.float32),
            pltpu.VMEM((_K, 128), jnp.float32),
            pltpu.VMEM((_K, 16), jnp.float32),
            pltpu.VMEM((_K, 16), jnp.float32),
            pltpu.VMEM((16,), jnp.float32),
            pltpu.VMEM((16,), jnp.float32),
            pltpu.SemaphoreType.DMA,
            pltpu.SemaphoreType.DMA,
        ],
    )
    return f(Rn, Cn, ridx, cidx, eawin, eawout, wi1, wo1, zeros_nd)


def kernel(x_s, x_t, edge_attr_s, edge_attr_t, edge_index_s, edge_index_t,
           x_s_batch, x_t_batch, y, node_emb, edge_emb, W_in_0, W_in_1,
           W_out_0, W_out_1, gin_W, gin_b, gout_W, gout_b, cls_W1, cls_b1,
           cls_W2, cls_b2):
    zeros_nd = jnp.zeros((_NPAD, 128), jnp.float32)

    def init_nodes(x, table):
        h = table[jnp.argmax(x, axis=1)]
        return jnp.concatenate(
            [h, jnp.zeros((_NPAD - _N, 128), jnp.float32)], axis=0)

    def init_edges(ea_half, table):
        he = table[jnp.argmax(ea_half, axis=1)]
        return jnp.concatenate(
            [he, jnp.zeros((_EPAD - _E, 16), jnp.float32)], axis=0)

    def pad_idx(ix):
        return jnp.concatenate(
            [ix.astype(jnp.int32),
             jnp.full((_EPAD - _E,), _N, jnp.int32)], axis=0)

    def dag(x0, ea_in, ea_out, ridx, cidx):
        x = x0
        acc2 = jnp.zeros((2 * _NPAD, 128), jnp.float32)
        bias = jnp.zeros((128,), jnp.float32)
        for i in range(_L):
            Wr = jnp.concatenate(
                [gin_W[i], W_in_0[i][:128], W_out_0[i][144:]], axis=1)
            Wc = jnp.concatenate(
                [gout_W[i], W_in_0[i][144:], W_out_0[i][:128]], axis=1)
            W = jnp.concatenate([Wr, Wc], axis=1)
            x, Rn, Cn = _proj(x, acc2, bias, W)
            eawin = _matmul(ea_in, W_in_0[i][128:144], 1024)
            eawout = _matmul(ea_out, W_out_0[i][128:144], 1024)
            acc2, ea_in, ea_out = _sc_sparse(
                Rn, Cn, ridx, cidx, eawin, eawout,
                W_in_1[i].reshape(16), W_out_1[i].reshape(16), zeros_nd)
            bias = gin_b[i] + gout_b[i]
        return (x + acc2[:_NPAD] + acc2[_NPAD:] + bias)[:_N]

    h_s = dag(init_nodes(x_s, node_emb),
              init_edges(edge_attr_s[:_E], edge_emb),
              init_edges(edge_attr_s[_E:], edge_emb),
              pad_idx(edge_index_s[0]), pad_idx(edge_index_s[1]))
    h_t = dag(init_nodes(x_t, node_emb),
              init_edges(edge_attr_t[:_E], edge_emb),
              init_edges(edge_attr_t[_E:], edge_emb),
              pad_idx(edge_index_t[0]), pad_idx(edge_index_t[1]))

    B = y.shape[0]

    def pool(x, batch_ids):
        sums = jax.ops.segment_sum(x, batch_ids, num_segments=B)
        cnt = jax.ops.segment_sum(jnp.ones((x.shape[0],), x.dtype), batch_ids,
                                  num_segments=B)
        return sums / jnp.maximum(cnt, 1.0)[:, None]

    g_s = pool(h_s, x_s_batch)
    g_t = pool(h_t, x_t_batch)
    z = jnp.concatenate([g_s, g_t], axis=1)
    h = jax.nn.relu(z @ cls_W1 + cls_b1)
    pred = h @ cls_W2 + cls_b2
    logp = jax.nn.log_softmax(pred, axis=1)
    loss = -jnp.mean(logp[jnp.arange(B), y])
    return loss


# speedup vs baseline: 1.9728x; 1.2231x over previous
import functools

import jax
import jax.numpy as jnp
from jax import lax
from jax.experimental import pallas as pl
from jax.experimental.pallas import tpu as pltpu
from jax.experimental.pallas import tpu_sc as plsc

_L = 2
_N = 10000
_E = 160000
_NPAD = 10112
_NT = 32
_EPT = 5184
_EPAD = _NT * _EPT
_K = 32
_NCH = _EPT // _K
_GRP = 6
_RPS = _NPAD // 16


def _matmul_body(x_ref, w_ref, o_ref):
    o_ref[...] = jnp.dot(x_ref[...], w_ref[...],
                         preferred_element_type=jnp.float32)


def _matmul(x, w, bn):
    n, k = x.shape
    m = w.shape[1]
    return pl.pallas_call(
        _matmul_body,
        grid=(n // bn,),
        in_specs=[pl.BlockSpec((bn, k), lambda i: (i, 0)),
                  pl.BlockSpec((k, m), lambda i: (0, 0))],
        out_specs=pl.BlockSpec((bn, m), lambda i: (i, 0)),
        out_shape=jax.ShapeDtypeStruct((n, m), jnp.float32),
    )(x, w)


def _proj_body(x_ref, acc_ref, b_ref, w_ref, xn_ref, rp_ref, cp_ref,
               rg_ref, cg_ref):
    xn = (x_ref[...] + acc_ref[0] + acc_ref[1] + b_ref[...])
    xn_ref[...] = xn
    rc = jnp.dot(xn, w_ref[...], preferred_element_type=jnp.float32)
    rp_ref[...] = rc[:, 0:128]
    cp_ref[...] = rc[:, 128:256]
    rg_ref[...] = rc[:, 256:288]
    cg_ref[...] = rc[:, 288:320]


def _proj(x, acc2, bias, w):
    bn = 632
    return pl.pallas_call(
        _proj_body,
        grid=(_NPAD // bn,),
        in_specs=[pl.BlockSpec((bn, 128), lambda i: (i, 0)),
                  pl.BlockSpec((2, bn, 128), lambda i: (0, i, 0)),
                  pl.BlockSpec((1, 128), lambda i: (0, 0)),
                  pl.BlockSpec((128, 320), lambda i: (0, 0))],
        out_specs=[pl.BlockSpec((bn, 128), lambda i: (i, 0)),
                   pl.BlockSpec((bn, 128), lambda i: (i, 0)),
                   pl.BlockSpec((bn, 128), lambda i: (i, 0)),
                   pl.BlockSpec((bn, 32), lambda i: (i, 0)),
                   pl.BlockSpec((bn, 32), lambda i: (i, 0))],
        out_shape=[jax.ShapeDtypeStruct((_NPAD, 128), jnp.float32),
                   jax.ShapeDtypeStruct((_NPAD, 128), jnp.float32),
                   jax.ShapeDtypeStruct((_NPAD, 128), jnp.float32),
                   jax.ShapeDtypeStruct((_NPAD, 32), jnp.float32),
                   jax.ShapeDtypeStruct((_NPAD, 32), jnp.float32)],
    )(x, acc2.reshape(2, _NPAD, 128), bias.reshape(1, 128), w)


def _sc_body(Rph, Cph, Rgh, Cgh, ri2h, ci2h, ein_h, eout_h, wih, woh, zh,
             acc_h, a2_h, b2_h,
             acc_s, ridx_t, cidx_t,
             rp_b, cp_b, rg_b, cg_b, ein_b, eout_b, av_b, bv_b,
             wiv, wov, sem_g, sem_s, sem_w):
    c = lax.axis_index("c")
    s = lax.axis_index("s")
    w = c * 16 + s
    pltpu.sync_copy(zh.at[pl.ds(s * _RPS, _RPS)],
                    acc_s.at[pl.ds(s * _RPS, _RPS)])
    pltpu.sync_copy(wih, wiv)
    pltpu.sync_copy(woh, wov)
    pltpu.sync_copy(ri2h.at[pl.ds(w * _NCH, _NCH)], ridx_t)
    pltpu.sync_copy(ci2h.at[pl.ds(w * _NCH, _NCH)], cidx_t)
    plsc.subcore_barrier()
    base = w * _EPT
    wir = wiv[...]
    wor = wov[...]

    def gathers(j, ps, sl):
        off = base + j * _K
        return (
            (Rph.at[ridx_t.at[j]], rp_b[ps], sem_g[ps]),
            (Cph.at[cidx_t.at[j]], cp_b[ps], sem_g[ps]),
            (Rgh.at[ridx_t.at[j]], rg_b[sl], sem_g[ps]),
            (Cgh.at[cidx_t.at[j]], cg_b[sl], sem_g[ps]),
            (ein_h.at[pl.ds(off, _K)], ein_b[sl], sem_g[ps]),
            (eout_h.at[pl.ds(off, _K)], eout_b[sl], sem_g[ps]),
        )

    def scatters(j, ps):
        return (
            (rp_b[ps], acc_s.at[cidx_t.at[j]], sem_s[ps]),
            (cp_b[ps], acc_s.at[ridx_t.at[j]], sem_s[ps]),
        )

    def writes(j, sl):
        off = base + j * _K
        return (
            (av_b[sl], a2_h.at[pl.ds(off, _K)], sem_w[sl]),
            (bv_b[sl], b2_h.at[pl.ds(off, _K)], sem_w[sl]),
        )

    def start_all(triples, add=False):
        for src, dst, sem in triples:
            pltpu.async_copy(src, dst, sem, add=add)

    def wait_all(triples):
        for src, dst, sem in triples:
            pltpu.make_async_copy(src, dst, sem).wait()

    def compute(ps, sl):
        rp, cp = rp_b[ps], cp_b[ps]
        rg, cg = rg_b[sl], cg_b[sl]
        ein, eout = ein_b[sl], eout_b[sl]
        av, bv = av_b[sl], bv_b[sl]

        def pair(t, carry):
            for u in range(2):
                e = t * 2 + u
                a = rg[e, pl.ds(0, 16)] + ein[e, :] + cg[e, pl.ds(0, 16)]
                b = rg[e, pl.ds(16, 16)] + eout[e, :] + cg[e, pl.ds(16, 16)]
                av[e, :] = a
                bv[e, :] = b
                zi = jnp.broadcast_to(jnp.sum(a * wir, axis=0), (16,))
                zo = jnp.broadcast_to(jnp.sum(b * wor, axis=0), (16,))
                gi = 1.0 / (1.0 + jnp.exp(-zi))
                go = 1.0 / (1.0 + jnp.exp(-zo))
                for q in range(8):
                    rp[e, pl.ds(q * 16, 16)] = gi * rp[e, pl.ds(q * 16, 16)]
                    cp[e, pl.ds(q * 16, 16)] = go * cp[e, pl.ds(q * 16, 16)]
            return carry

        lax.fori_loop(0, _K // 2, pair, 0)

    start_all(gathers(0, 0, 0))

    def ring(g, carry):
        for u in range(_GRP):
            j = g * _GRP + u
            ps, sl = u % 3, u % 2
            psm2 = (u - 2) % 3
            psp1, slp1 = (u + 1) % 3, (u + 1) % 2

            @pl.when(j >= 2)
            def _():
                wait_all(scatters(j - 2, psm2))
                wait_all(writes(j - 2, sl))

            @pl.when(j + 1 < _NCH)
            def _():
                start_all(gathers(j + 1, psp1, slp1))

            wait_all(gathers(j, ps, sl))
            compute(ps, sl)
            start_all(scatters(j, ps), add=True)
            start_all(writes(j, sl))
        return carry

    lax.fori_loop(0, _NCH // _GRP, ring, 0)
    for j in (_NCH - 2, _NCH - 1):
        u = j % _GRP
        wait_all(scatters(j, u % 3))
        wait_all(writes(j, u % 2))
    plsc.subcore_barrier()
    pltpu.sync_copy(acc_s.at[pl.ds(s * _RPS, _RPS)],
                    acc_h.at[pl.ds(c * _NPAD + s * _RPS, _RPS)])


def _sc_sparse(Rp, Cp, Rgt, Cgt, ridx2, cidx2, eawin, eawout, wi1, wo1,
               zeros_nd):
    mesh = plsc.VectorSubcoreMesh(core_axis_name="c", subcore_axis_name="s")
    f = pl.kernel(
        _sc_body,
        out_type=(jax.ShapeDtypeStruct((2 * _NPAD, 128), jnp.float32),
                  jax.ShapeDtypeStruct((_EPAD, 16), jnp.float32),
                  jax.ShapeDtypeStruct((_EPAD, 16), jnp.float32)),
        mesh=mesh,
        compiler_params=pltpu.CompilerParams(needs_layout_passes=False,
                                             use_tc_tiling_on_sc=False),
        scratch_types=[
            pltpu.VMEM_SHARED((_NPAD, 128), jnp.float32),
            pltpu.VMEM((_NCH, _K), jnp.int32),
            pltpu.VMEM((_NCH, _K), jnp.int32),
            [pltpu.VMEM((_K, 128), jnp.float32) for _ in range(3)],
            [pltpu.VMEM((_K, 128), jnp.float32) for _ in range(3)],
            [pltpu.VMEM((_K, 32), jnp.float32) for _ in range(2)],
            [pltpu.VMEM((_K, 32), jnp.float32) for _ in range(2)],
            [pltpu.VMEM((_K, 16), jnp.float32) for _ in range(2)],
            [pltpu.VMEM((_K, 16), jnp.float32) for _ in range(2)],
            [pltpu.VMEM((_K, 16), jnp.float32) for _ in range(2)],
            [pltpu.VMEM((_K, 16), jnp.float32) for _ in range(2)],
            pltpu.VMEM((16,), jnp.float32),
            pltpu.VMEM((16,), jnp.float32),
            [pltpu.SemaphoreType.DMA for _ in range(3)],
            [pltpu.SemaphoreType.DMA for _ in range(3)],
            [pltpu.SemaphoreType.DMA for _ in range(2)],
        ],
    )
    return f(Rp, Cp, Rgt, Cgt, ridx2, cidx2, eawin, eawout, wi1, wo1,
             zeros_nd)


def kernel(x_s, x_t, edge_attr_s, edge_attr_t, edge_index_s, edge_index_t,
           x_s_batch, x_t_batch, y, node_emb, edge_emb, W_in_0, W_in_1,
           W_out_0, W_out_1, gin_W, gin_b, gout_W, gout_b, cls_W1, cls_b1,
           cls_W2, cls_b2):
    zeros_nd = jnp.zeros((_NPAD, 128), jnp.float32)

    def init_nodes(x, table):
        h = table[jnp.argmax(x, axis=1)]
        return jnp.concatenate(
            [h, jnp.zeros((_NPAD - _N, 128), jnp.float32)], axis=0)

    def init_edges(ea_half, table):
        he = table[jnp.argmax(ea_half, axis=1)]
        return jnp.concatenate(
            [he, jnp.zeros((_EPAD - _E, 16), jnp.float32)], axis=0)

    def pad_idx(ix):
        return jnp.concatenate(
            [ix.astype(jnp.int32),
             jnp.full((_EPAD - _E,), _N, jnp.int32)],
            axis=0).reshape(_NT * _NCH, _K)

    def dag(x0, ea_in, ea_out, ridx2, cidx2):
        x = x0
        acc2 = jnp.zeros((2 * _NPAD, 128), jnp.float32)
        bias = jnp.zeros((128,), jnp.float32)
        for i in range(_L):
            W = jnp.concatenate(
                [gin_W[i], gout_W[i], W_in_0[i][:128], W_out_0[i][144:],
                 W_in_0[i][144:], W_out_0[i][:128]], axis=1)
            x, Rp, Cp, Rgt, Cgt = _proj(x, acc2, bias, W)
            eawin = _matmul(ea_in, W_in_0[i][128:144], 1024)
            eawout = _matmul(ea_out, W_out_0[i][128:144], 1024)
            acc2, ea_in, ea_out = _sc_sparse(
                Rp, Cp, Rgt, Cgt, ridx2, cidx2, eawin, eawout,
                W_in_1[i].reshape(16), W_out_1[i].reshape(16), zeros_nd)
            bias = gin_b[i] + gout_b[i]
        return (x + acc2[:_NPAD] + acc2[_NPAD:] + bias)[:_N]

    h_s = dag(init_nodes(x_s, node_emb),
              init_edges(edge_attr_s[:_E], edge_emb),
              init_edges(edge_attr_s[_E:], edge_emb),
              pad_idx(edge_index_s[0]), pad_idx(edge_index_s[1]))
    h_t = dag(init_nodes(x_t, node_emb),
              init_edges(edge_attr_t[:_E], edge_emb),
              init_edges(edge_attr_t[_E:], edge_emb),
              pad_idx(edge_index_t[0]), pad_idx(edge_index_t[1]))

    B = y.shape[0]

    def pool(x, batch_ids):
        sums = jax.ops.segment_sum(x, batch_ids, num_segments=B)
        cnt = jax.ops.segment_sum(jnp.ones((x.shape[0],), x.dtype), batch_ids,
                                  num_segments=B)
        return sums / jnp.maximum(cnt, 1.0)[:, None]

    g_s = pool(h_s, x_s_batch)
    g_t = pool(h_t, x_t_batch)
    z = jnp.concatenate([g_s, g_t], axis=1)
    h = jax.nn.relu(z @ cls_W1 + cls_b1)
    pred = h @ cls_W2 + cls_b2
    logp = jax.nn.log_softmax(pred, axis=1)
    loss = -jnp.mean(logp[jnp.arange(B), y])
    return loss
